# Initial kernel scaffold; baseline (speedup 1.0000x reference)
import jax
import jax.numpy as jnp
from jax.experimental import pallas as pl


def kernel(x, codebook):
    raise NotImplementedError("write your pallas kernel here")


# speedup vs baseline: 6.3894x; 6.3894x over previous
import jax
import jax.numpy as jnp
from jax.experimental import pallas as pl

N = 16384
D = 256
M = 8
NCODES = 256
DSUB = D // M
BN = 2048


def _pdt_body(xb_ref, cbt_ref, out_ref):
    xb = xb_ref[:]
    xnorm = jnp.sum(xb * xb, axis=-1)
    sum_d = jnp.zeros((BN,), jnp.float32)
    sum_dot = jnp.zeros((BN,), jnp.float32)
    for m in range(M):
        xc = xb[:, m * DSUB:(m + 1) * DSUB]
        cbt = cbt_ref[m]
        cbn = jnp.sum(cbt * cbt, axis=0)
        s = cbn[None, :] - 2.0 * jnp.dot(xc, cbt,
                                         preferred_element_type=jnp.float32)
        smin = jnp.min(s, axis=-1)
        mask = s == smin[:, None]
        cbn_sel = jnp.max(jnp.where(mask, cbn[None, :], -jnp.inf), axis=-1)
        xcn = jnp.sum(xc * xc, axis=-1)
        sum_d = sum_d + jnp.maximum(xcn + smin, 0.0)
        sum_dot = sum_dot + 0.5 * (cbn_sel - smin)
    out_ref[:] = jnp.sqrt(sum_d) + jnp.abs(sum_dot - xnorm)


def kernel(x, codebook):
    cbt = jnp.transpose(codebook, (0, 2, 1))
    return pl.pallas_call(
        _pdt_body,
        grid=(N // BN,),
        in_specs=[
            pl.BlockSpec((BN, D), lambda i: (i, 0)),
            pl.BlockSpec((M, DSUB, NCODES), lambda i: (0, 0, 0)),
        ],
        out_specs=pl.BlockSpec((BN,), lambda i: (i,)),
        out_shape=jax.ShapeDtypeStruct((N,), jnp.float32),
    )(x, cbt)
